# baseline (speedup 1.0000x reference)
import functools

import jax
import jax.numpy as jnp
from jax import lax
from jax.experimental import pallas as pl
from jax.experimental.pallas import tpu as pltpu
from jax.experimental.pallas import tpu_sc as plsc

N = 10000
E = 160000
N_PAD = 10240

NC = 2
NS = 16
EC = 128
NCHUNK = E // EC
RPT = N_PAD // NS
RCHUNK = RPT // EC

BE = 3200
BN = 1000


def _elu(v):
    return jnp.where(v > 0, v, jnp.exp(v) - 1.0)


def _edge_mlp_body(ew_ref, m1_ref, b1_ref, m2_ref, b2_ref, m3_ref, b3_ref,
                   out_ref):
    t = _elu(jnp.dot(ew_ref[...], m1_ref[...],
                     preferred_element_type=jnp.float32) + b1_ref[...])
    t = _elu(jnp.dot(t.astype(jnp.bfloat16), m2_ref[...],
                     preferred_element_type=jnp.float32) + b2_ref[...])
    o = (jnp.dot(t, m3_ref[...],
                 preferred_element_type=jnp.float32) + b3_ref[...])
    out_ref[0] = o[:, 0]
    out_ref[1] = o[:, 1]


def _edge_mlp(ew, m1, b1, m2, b2, m3, b3):
    full = lambda a: pl.BlockSpec(a.shape, lambda i: (0,) * a.ndim)
    return pl.pallas_call(
        _edge_mlp_body,
        grid=(E // BE,),
        in_specs=[pl.BlockSpec((BE, 2), lambda i: (i, 0)),
                  full(m1), full(b1), full(m2), full(b2), full(m3), full(b3)],
        out_specs=pl.BlockSpec((2, BE), lambda i: (0, i)),
        out_shape=jax.ShapeDtypeStruct((2, E), jnp.float32),
    )(ew, m1, b1, m2, b2, m3, b3)


def _h0_body(x_ref, w1_ref, w2_ref, out_ref):
    xb = x_ref[...].astype(jnp.bfloat16)
    h1 = jnp.dot(xb, w1_ref[...], preferred_element_type=jnp.float32)
    h2 = jnp.dot(xb, w2_ref[...], preferred_element_type=jnp.float32)
    out_ref[0] = h1[:, :128]
    out_ref[1] = h1[:, 128:]
    out_ref[2] = h2[:, :128]
    out_ref[3] = h2[:, 128:]


def _h0(x, w1, w2):
    full = lambda a: pl.BlockSpec(a.shape, lambda i: (0,) * a.ndim)
    return pl.pallas_call(
        _h0_body,
        grid=(N // BN,),
        in_specs=[pl.BlockSpec((BN, 256), lambda i: (i, 0)),
                  full(w1), full(w2)],
        out_specs=pl.BlockSpec((4, BN, 128), lambda i: (0, i, 0)),
        out_shape=jax.ShapeDtypeStruct((4, N, 128), jnp.float32),
    )(x, w1, w2)


def _mid_body(agg_ref, b1_ref, b2_ref, w1_ref, w2_ref, out_ref):
    y1 = jnp.maximum(
        jnp.concatenate([agg_ref[0], agg_ref[1]], axis=1) + b1_ref[...], 0.0)
    y2 = jnp.maximum(
        jnp.concatenate([agg_ref[2], agg_ref[3]], axis=1) + b2_ref[...], 0.0)
    y = jnp.concatenate([y1, y2], axis=1).astype(jnp.bfloat16)
    h1 = jnp.dot(y, w1_ref[...], preferred_element_type=jnp.float32)
    h2 = jnp.dot(y, w2_ref[...], preferred_element_type=jnp.float32)
    out_ref[0] = h1[:, :128]
    out_ref[1] = h1[:, 128:]
    out_ref[2] = h2[:, :128]
    out_ref[3] = h2[:, 128:]


def _mid(agg, b1, b2, w1, w2):
    full = lambda a: pl.BlockSpec(a.shape, lambda i: (0,) * a.ndim)
    return pl.pallas_call(
        _mid_body,
        grid=(N // BN,),
        in_specs=[pl.BlockSpec((4, BN, 128), lambda i: (0, i, 0)),
                  full(b1), full(b2), full(w1), full(w2)],
        out_specs=pl.BlockSpec((4, BN, 128), lambda i: (0, i, 0)),
        out_shape=jax.ShapeDtypeStruct((4, N, 128), jnp.float32),
    )(agg, b1, b2, w1, w2)


def _readout_body(agg_ref, b1_ref, b2_ref, r1_ref, rb1_ref, r2_ref, rb2_ref,
                  r3_ref, rb3_ref, out_ref):
    y1 = jnp.maximum(
        jnp.concatenate([agg_ref[0], agg_ref[1]], axis=1) + b1_ref[...], 0.0)
    y2 = jnp.maximum(
        jnp.concatenate([agg_ref[2], agg_ref[3]], axis=1) + b2_ref[...], 0.0)
    y = jnp.concatenate([y1, y2], axis=1).astype(jnp.bfloat16)
    r = _elu(jnp.dot(y, r1_ref[...],
                     preferred_element_type=jnp.float32) + rb1_ref[...])
    r = _elu(jnp.dot(r.astype(jnp.bfloat16), r2_ref[...],
                     preferred_element_type=jnp.float32) + rb2_ref[...])
    out_ref[...] = (jnp.dot(r.astype(jnp.bfloat16), r3_ref[...],
                            preferred_element_type=jnp.float32) + rb3_ref[...])


def _readout(agg, b1, b2, r1, rb1, r2, rb2, r3, rb3):
    full = lambda a: pl.BlockSpec(a.shape, lambda i: (0,) * a.ndim)
    return pl.pallas_call(
        _readout_body,
        grid=(N // BN,),
        in_specs=[pl.BlockSpec((4, BN, 128), lambda i: (0, i, 0)),
                  full(b1), full(b2), full(r1), full(rb1), full(r2), full(rb2),
                  full(r3), full(rb3)],
        out_specs=pl.BlockSpec((BN, 128), lambda i: (i, 0)),
        out_shape=jax.ShapeDtypeStruct((N, 128), jnp.float32),
    )(agg, b1, b2, r1, rb1, r2, rb2, r3, rb3)


def _sc_agg_body(h_hbm, src_hbm, dst_hbm, ew_hbm, out_hbm,
                 src_0, dst_0, ew_0, src_1, dst_1, ew_1,
                 src_2, dst_2, ew_2, src_3, dst_3, ew_3,
                 rows_a, rows_b, acc_sh,
                 isem_0, isem_1, isem_2, isem_3, gsem_a, gsem_b):
    c = lax.axis_index("c")
    s = lax.axis_index("s")
    zero = jnp.zeros((16,), jnp.float32)
    isets = ((src_0, dst_0, ew_0, isem_0), (src_1, dst_1, ew_1, isem_1),
             (src_2, dst_2, ew_2, isem_2), (src_3, dst_3, ew_3, isem_3))
    gslots = ((rows_a, gsem_a), (rows_b, gsem_b))

    n_s = jnp.where(s < NCHUNK % NS, NCHUNK // NS + 1, NCHUNK // NS)
    n_body = (NCHUNK // NS + 1 + 3) // 4

    def zero_row(i, carry):
        for j in range(8):
            rows_a[i, pl.ds(j * 16, 16)] = zero
        return carry

    for k in range(2):
        def idx_copies(q, iset):
            src_v, dst_v, ew_v, isem = iset
            off = (s + q * NS) * EC
            return (
                pltpu.make_async_copy(src_hbm.at[pl.ds(off, EC)], src_v, isem),
                pltpu.make_async_copy(dst_hbm.at[pl.ds(off, EC)], dst_v, isem),
                pltpu.make_async_copy(ew_hbm.at[c].at[pl.ds(off, EC)], ew_v,
                                      isem),
            )

        def load_idx(q, iset):
            for cp in idx_copies(q, iset):
                cp.start()

        def wait_idx_issue_gather(q, iset, gslot):
            for cp in idx_copies(q, iset):
                cp.wait()
            rows_v, gsem = gslot
            pltpu.async_copy(h_hbm.at[2 * c + k].at[iset[0]], rows_v, gsem)

        def wait_scale_scatter(q, iset, gslot):
            src_v, dst_v, ew_v, _ = iset
            rows_v, gsem = gslot
            pltpu.make_async_copy(h_hbm.at[2 * c + k].at[src_v],
                                  rows_v, gsem).wait()

            @plsc.parallel_loop(0, EC // 16, step=1, unroll=4)
            def _(g):
                wg = ew_v[pl.ds(g * 16, 16)]
                for l in range(16):
                    w = jnp.full((16,), wg[l])
                    row = g * 16 + l
                    for j in range(8):
                        rows_v[row, pl.ds(j * 16, 16)] = (
                            rows_v[row, pl.ds(j * 16, 16)] * w)

            pltpu.sync_copy(rows_v, acc_sh.at[dst_v], add=True)

        lax.fori_loop(0, EC, zero_row, 0)
        for r in range(RCHUNK):
            pltpu.sync_copy(rows_a, acc_sh.at[pl.ds(s * RPT + r * EC, EC)])
        plsc.subcore_barrier()

        for q in range(4):
            load_idx(q, isets[q])
        wait_idx_issue_gather(0, isets[0], gslots[0])
        wait_idx_issue_gather(1, isets[1], gslots[1])

        def quad_body(u, carry):
            q0 = 4 * u
            for r in range(4):
                q = q0 + r
                iset, gslot = isets[r], gslots[r % 2]
                iset2 = isets[(r + 2) % 4]

                @pl.when(q < n_s)
                def _():
                    wait_scale_scatter(q, iset, gslot)

                @pl.when(q + 2 < n_s)
                def _():
                    wait_idx_issue_gather(q + 2, iset2, gslot)

                @pl.when(q + 4 < n_s)
                def _():
                    load_idx(q + 4, iset)

            return carry

        lax.fori_loop(0, n_body, quad_body, 0)
        plsc.subcore_barrier()

        pltpu.sync_copy(acc_sh.at[pl.ds(s * RPT, RPT)],
                        out_hbm.at[2 * c + k].at[pl.ds(s * RPT, RPT)])
        plsc.subcore_barrier()


_sc_agg = pl.kernel(
    _sc_agg_body,
    out_type=jax.ShapeDtypeStruct((4, N_PAD, 128), jnp.float32),
    mesh=plsc.VectorSubcoreMesh(core_axis_name="c", subcore_axis_name="s",
                                num_cores=NC, num_subcores=NS),
    scratch_types=(
        [t for _ in range(4)
         for t in (pltpu.VMEM((EC,), jnp.int32),
                   pltpu.VMEM((EC,), jnp.int32),
                   pltpu.VMEM((EC,), jnp.float32))]
        + [
            pltpu.VMEM((EC, 128), jnp.float32),
            pltpu.VMEM((EC, 128), jnp.float32),
            pltpu.VMEM_SHARED((N_PAD, 128), jnp.float32),
            pltpu.SemaphoreType.DMA,
            pltpu.SemaphoreType.DMA,
            pltpu.SemaphoreType.DMA,
            pltpu.SemaphoreType.DMA,
            pltpu.SemaphoreType.DMA,
            pltpu.SemaphoreType.DMA,
        ]),
)


def kernel(x, edge_index, edge_weight, W01, b01, W02, b02, W11, b11, W12, b12,
           W21, b21, W22, b22, R1, rb1, R2, rb2, R3, rb3, Em1, eb1, Em2, eb2,
           Em3, eb3):
    src = edge_index[0]
    dst = edge_index[1]

    bf = lambda a: a.astype(jnp.bfloat16)
    ew = _edge_mlp(edge_weight, Em1, eb1, bf(Em2), eb2, Em3, eb3)

    h = _h0(x, bf(W01), bf(W02))
    agg = _sc_agg(h, src, dst, ew)
    h = _mid(agg, b01, b02, bf(W11), bf(W12))
    agg = _sc_agg(h, src, dst, ew)
    h = _mid(agg, b11, b12, bf(W21), bf(W22))
    agg = _sc_agg(h, src, dst, ew)
    return _readout(agg, b21, b22, bf(R1), rb1, bf(R2), rb2, bf(R3), rb3)


# speedup vs baseline: 1.0908x; 1.0908x over previous
import functools

import jax
import jax.numpy as jnp
from jax import lax
from jax.experimental import pallas as pl
from jax.experimental.pallas import tpu as pltpu
from jax.experimental.pallas import tpu_sc as plsc

N = 10000
E = 160000
N_PAD = 10240

NC = 2
NS = 16
EC = 128
NCHUNK = E // EC
RPT = N_PAD // NS
RCHUNK = RPT // EC

BE = 3200
BN = 1000


def _elu(v):
    return jnp.where(v > 0, v, jnp.exp(v) - 1.0)


def _edge_mlp_body(ew_ref, m1_ref, b1_ref, m2_ref, b2_ref, m3_ref, b3_ref,
                   out_ref):
    t = _elu(jnp.dot(ew_ref[...], m1_ref[...],
                     preferred_element_type=jnp.float32) + b1_ref[...])
    t = _elu(jnp.dot(t.astype(jnp.bfloat16), m2_ref[...],
                     preferred_element_type=jnp.float32) + b2_ref[...])
    o = (jnp.dot(t, m3_ref[...],
                 preferred_element_type=jnp.float32) + b3_ref[...])
    out_ref[0] = o[:, 0]
    out_ref[1] = o[:, 1]


def _edge_mlp(ew, m1, b1, m2, b2, m3, b3):
    full = lambda a: pl.BlockSpec(a.shape, lambda i: (0,) * a.ndim)
    return pl.pallas_call(
        _edge_mlp_body,
        grid=(E // BE,),
        in_specs=[pl.BlockSpec((BE, 2), lambda i: (i, 0)),
                  full(m1), full(b1), full(m2), full(b2), full(m3), full(b3)],
        out_specs=pl.BlockSpec((2, BE), lambda i: (0, i)),
        out_shape=jax.ShapeDtypeStruct((2, E), jnp.float32),
    )(ew, m1, b1, m2, b2, m3, b3)


def _h0_body(x_ref, w1_ref, w2_ref, out_ref):
    xb = x_ref[...].astype(jnp.bfloat16)
    h1 = jnp.dot(xb, w1_ref[...], preferred_element_type=jnp.float32)
    h2 = jnp.dot(xb, w2_ref[...], preferred_element_type=jnp.float32)
    out_ref[0] = h1[:, :128]
    out_ref[1] = h1[:, 128:]
    out_ref[2] = h2[:, :128]
    out_ref[3] = h2[:, 128:]


def _h0(x, w1, w2):
    full = lambda a: pl.BlockSpec(a.shape, lambda i: (0,) * a.ndim)
    return pl.pallas_call(
        _h0_body,
        grid=(N // BN,),
        in_specs=[pl.BlockSpec((BN, 256), lambda i: (i, 0)),
                  full(w1), full(w2)],
        out_specs=pl.BlockSpec((4, BN, 128), lambda i: (0, i, 0)),
        out_shape=jax.ShapeDtypeStruct((4, N, 128), jnp.float32),
    )(x, w1, w2)


def _mid_body(agg_ref, b1_ref, b2_ref, w1_ref, w2_ref, out_ref):
    y1 = jnp.maximum(
        jnp.concatenate([agg_ref[0], agg_ref[1]], axis=1) + b1_ref[...], 0.0)
    y2 = jnp.maximum(
        jnp.concatenate([agg_ref[2], agg_ref[3]], axis=1) + b2_ref[...], 0.0)
    y = jnp.concatenate([y1, y2], axis=1).astype(jnp.bfloat16)
    h1 = jnp.dot(y, w1_ref[...], preferred_element_type=jnp.float32)
    h2 = jnp.dot(y, w2_ref[...], preferred_element_type=jnp.float32)
    out_ref[0] = h1[:, :128]
    out_ref[1] = h1[:, 128:]
    out_ref[2] = h2[:, :128]
    out_ref[3] = h2[:, 128:]


def _mid(agg, b1, b2, w1, w2):
    full = lambda a: pl.BlockSpec(a.shape, lambda i: (0,) * a.ndim)
    return pl.pallas_call(
        _mid_body,
        grid=(N // BN,),
        in_specs=[pl.BlockSpec((4, BN, 128), lambda i: (0, i, 0)),
                  full(b1), full(b2), full(w1), full(w2)],
        out_specs=pl.BlockSpec((4, BN, 128), lambda i: (0, i, 0)),
        out_shape=jax.ShapeDtypeStruct((4, N, 128), jnp.float32),
    )(agg, b1, b2, w1, w2)


def _readout_body(agg_ref, b1_ref, b2_ref, r1_ref, rb1_ref, r2_ref, rb2_ref,
                  r3_ref, rb3_ref, out_ref):
    y1 = jnp.maximum(
        jnp.concatenate([agg_ref[0], agg_ref[1]], axis=1) + b1_ref[...], 0.0)
    y2 = jnp.maximum(
        jnp.concatenate([agg_ref[2], agg_ref[3]], axis=1) + b2_ref[...], 0.0)
    y = jnp.concatenate([y1, y2], axis=1).astype(jnp.bfloat16)
    r = _elu(jnp.dot(y, r1_ref[...],
                     preferred_element_type=jnp.float32) + rb1_ref[...])
    r = _elu(jnp.dot(r.astype(jnp.bfloat16), r2_ref[...],
                     preferred_element_type=jnp.float32) + rb2_ref[...])
    out_ref[...] = (jnp.dot(r.astype(jnp.bfloat16), r3_ref[...],
                            preferred_element_type=jnp.float32) + rb3_ref[...])


def _readout(agg, b1, b2, r1, rb1, r2, rb2, r3, rb3):
    full = lambda a: pl.BlockSpec(a.shape, lambda i: (0,) * a.ndim)
    return pl.pallas_call(
        _readout_body,
        grid=(N // BN,),
        in_specs=[pl.BlockSpec((4, BN, 128), lambda i: (0, i, 0)),
                  full(b1), full(b2), full(r1), full(rb1), full(r2), full(rb2),
                  full(r3), full(rb3)],
        out_specs=pl.BlockSpec((BN, 128), lambda i: (i, 0)),
        out_shape=jax.ShapeDtypeStruct((N, 128), jnp.float32),
    )(agg, b1, b2, r1, rb1, r2, rb2, r3, rb3)


def _sc_agg_body(h_hbm, src_hbm, dst_hbm, ew_hbm, out_hbm,
                 src_0, dst_0, ew_0, src_1, dst_1, ew_1,
                 src_2, dst_2, ew_2, src_3, dst_3, ew_3,
                 rows_a, rows_b, acc_sh,
                 isem_0, isem_1, isem_2, isem_3, gsem_a, gsem_b):
    c = lax.axis_index("c")
    s = lax.axis_index("s")
    zero = jnp.zeros((16,), jnp.float32)
    isets = ((src_0, dst_0, ew_0, isem_0), (src_1, dst_1, ew_1, isem_1),
             (src_2, dst_2, ew_2, isem_2), (src_3, dst_3, ew_3, isem_3))
    gslots = ((rows_a, gsem_a), (rows_b, gsem_b))

    n_s = jnp.where(s < NCHUNK % NS, NCHUNK // NS + 1, NCHUNK // NS)
    n_body = (NCHUNK // NS + 1 + 3) // 4

    def zero_row(i, carry):
        for j in range(8):
            rows_a[i, pl.ds(j * 16, 16)] = zero
        return carry

    for k in range(2):
        def idx_copies(q, iset):
            src_v, dst_v, ew_v, isem = iset
            off = (s + q * NS) * EC
            return (
                pltpu.make_async_copy(src_hbm.at[pl.ds(off, EC)], src_v, isem),
                pltpu.make_async_copy(dst_hbm.at[pl.ds(off, EC)], dst_v, isem),
                pltpu.make_async_copy(ew_hbm.at[c].at[pl.ds(off, EC)], ew_v,
                                      isem),
            )

        def load_idx(q, iset):
            for cp in idx_copies(q, iset):
                cp.start()

        def wait_idx_issue_gather(q, iset, gslot):
            for cp in idx_copies(q, iset):
                cp.wait()
            rows_v, gsem = gslot
            pltpu.async_copy(h_hbm.at[2 * c + k].at[iset[0]], rows_v, gsem)

        def wait_scale_scatter(q, iset, gslot):
            src_v, dst_v, ew_v, _ = iset
            rows_v, gsem = gslot
            pltpu.make_async_copy(h_hbm.at[2 * c + k].at[src_v],
                                  rows_v, gsem).wait()

            @plsc.parallel_loop(0, EC // 16, step=1, unroll=2)
            def _(g):
                wg = ew_v[pl.ds(g * 16, 16)]
                for l in range(16):
                    w = jnp.full((16,), wg[l])
                    row = g * 16 + l
                    for j in range(8):
                        rows_v[row, pl.ds(j * 16, 16)] = (
                            rows_v[row, pl.ds(j * 16, 16)] * w)

            pltpu.sync_copy(rows_v, acc_sh.at[dst_v], add=True)

        lax.fori_loop(0, EC, zero_row, 0)
        for r in range(RCHUNK):
            pltpu.sync_copy(rows_a, acc_sh.at[pl.ds(s * RPT + r * EC, EC)])
        plsc.subcore_barrier()

        for q in range(4):
            load_idx(q, isets[q])
        wait_idx_issue_gather(0, isets[0], gslots[0])
        wait_idx_issue_gather(1, isets[1], gslots[1])

        def quad_body(u, carry):
            q0 = 4 * u
            for r in range(4):
                q = q0 + r
                iset, gslot = isets[r], gslots[r % 2]
                iset2 = isets[(r + 2) % 4]

                @pl.when(q < n_s)
                def _():
                    wait_scale_scatter(q, iset, gslot)

                @pl.when(q + 2 < n_s)
                def _():
                    wait_idx_issue_gather(q + 2, iset2, gslot)

                @pl.when(q + 4 < n_s)
                def _():
                    load_idx(q + 4, iset)

            return carry

        lax.fori_loop(0, n_body, quad_body, 0)
        plsc.subcore_barrier()

        pltpu.sync_copy(acc_sh.at[pl.ds(s * RPT, RPT)],
                        out_hbm.at[2 * c + k].at[pl.ds(s * RPT, RPT)])
        plsc.subcore_barrier()


_sc_agg = pl.kernel(
    _sc_agg_body,
    out_type=jax.ShapeDtypeStruct((4, N_PAD, 128), jnp.float32),
    mesh=plsc.VectorSubcoreMesh(core_axis_name="c", subcore_axis_name="s",
                                num_cores=NC, num_subcores=NS),
    scratch_types=(
        [t for _ in range(4)
         for t in (pltpu.VMEM((EC,), jnp.int32),
                   pltpu.VMEM((EC,), jnp.int32),
                   pltpu.VMEM((EC,), jnp.float32))]
        + [
            pltpu.VMEM((EC, 128), jnp.float32),
            pltpu.VMEM((EC, 128), jnp.float32),
            pltpu.VMEM_SHARED((N_PAD, 128), jnp.float32),
            pltpu.SemaphoreType.DMA,
            pltpu.SemaphoreType.DMA,
            pltpu.SemaphoreType.DMA,
            pltpu.SemaphoreType.DMA,
            pltpu.SemaphoreType.DMA,
            pltpu.SemaphoreType.DMA,
        ]),
)


def kernel(x, edge_index, edge_weight, W01, b01, W02, b02, W11, b11, W12, b12,
           W21, b21, W22, b22, R1, rb1, R2, rb2, R3, rb3, Em1, eb1, Em2, eb2,
           Em3, eb3):
    src = edge_index[0]
    dst = edge_index[1]

    bf = lambda a: a.astype(jnp.bfloat16)
    ew = _edge_mlp(edge_weight, Em1, eb1, bf(Em2), eb2, Em3, eb3)

    h = _h0(x, bf(W01), bf(W02))
    agg = _sc_agg(h, src, dst, ew)
    h = _mid(agg, b01, b02, bf(W11), bf(W12))
    agg = _sc_agg(h, src, dst, ew)
    h = _mid(agg, b11, b12, bf(W21), bf(W22))
    agg = _sc_agg(h, src, dst, ew)
    return _readout(agg, b21, b22, bf(R1), rb1, bf(R2), rb2, bf(R3), rb3)


# speedup vs baseline: 1.0935x; 1.0025x over previous
import jax
import jax.numpy as jnp
from jax import lax
from jax.experimental import pallas as pl
from jax.experimental.pallas import tpu as pltpu
from jax.experimental.pallas import tpu_sc as plsc

N = 10000
E = 160000
N_PAD = 10240

NC = 2
NS = 16
EC = 128
NCHUNK = E // EC
RPT = N_PAD // NS
RCHUNK = RPT // EC

BE = 3200
BN = 1000


def _elu(v):
    return jnp.where(v > 0, v, jnp.exp(v) - 1.0)


def _edge_mlp_body(ew_ref, m1_ref, b1_ref, m2_ref, b2_ref, m3_ref, b3_ref,
                   out_ref):
    t = _elu(jnp.dot(ew_ref[...], m1_ref[...],
                     preferred_element_type=jnp.float32) + b1_ref[...])
    t = _elu(jnp.dot(t.astype(jnp.bfloat16), m2_ref[...],
                     preferred_element_type=jnp.float32) + b2_ref[...])
    o = (jnp.dot(t, m3_ref[...],
                 preferred_element_type=jnp.float32) + b3_ref[...])
    out_ref[0] = o[:, 0]
    out_ref[1] = o[:, 1]


def _edge_mlp(ew, m1, b1, m2, b2, m3, b3):
    full = lambda a: pl.BlockSpec(a.shape, lambda i: (0,) * a.ndim)
    return pl.pallas_call(
        _edge_mlp_body,
        grid=(E // BE,),
        in_specs=[pl.BlockSpec((BE, 2), lambda i: (i, 0)),
                  full(m1), full(b1), full(m2), full(b2), full(m3), full(b3)],
        out_specs=pl.BlockSpec((2, BE), lambda i: (0, i)),
        out_shape=jax.ShapeDtypeStruct((2, E), jnp.float32),
    )(ew, m1, b1, m2, b2, m3, b3)


def _h0_body(x_ref, w1_ref, w2_ref, out_ref):
    xb = x_ref[...].astype(jnp.bfloat16)
    h1 = jnp.dot(xb, w1_ref[...], preferred_element_type=jnp.float32)
    h2 = jnp.dot(xb, w2_ref[...], preferred_element_type=jnp.float32)
    out_ref[0] = h1[:, :128]
    out_ref[1] = h1[:, 128:]
    out_ref[2] = h2[:, :128]
    out_ref[3] = h2[:, 128:]


def _h0(x, w1, w2):
    full = lambda a: pl.BlockSpec(a.shape, lambda i: (0,) * a.ndim)
    return pl.pallas_call(
        _h0_body,
        grid=(N // BN,),
        in_specs=[pl.BlockSpec((BN, 256), lambda i: (i, 0)),
                  full(w1), full(w2)],
        out_specs=pl.BlockSpec((4, BN, 128), lambda i: (0, i, 0)),
        out_shape=jax.ShapeDtypeStruct((4, N, 128), jnp.float32),
    )(x, w1, w2)


def _mid_body(agg_ref, b1_ref, b2_ref, w1_ref, w2_ref, out_ref):
    y1 = jnp.maximum(
        jnp.concatenate([agg_ref[0], agg_ref[1]], axis=1) + b1_ref[...], 0.0)
    y2 = jnp.maximum(
        jnp.concatenate([agg_ref[2], agg_ref[3]], axis=1) + b2_ref[...], 0.0)
    y = jnp.concatenate([y1, y2], axis=1).astype(jnp.bfloat16)
    h1 = jnp.dot(y, w1_ref[...], preferred_element_type=jnp.float32)
    h2 = jnp.dot(y, w2_ref[...], preferred_element_type=jnp.float32)
    out_ref[0] = h1[:, :128]
    out_ref[1] = h1[:, 128:]
    out_ref[2] = h2[:, :128]
    out_ref[3] = h2[:, 128:]


def _mid(agg, b1, b2, w1, w2):
    full = lambda a: pl.BlockSpec(a.shape, lambda i: (0,) * a.ndim)
    return pl.pallas_call(
        _mid_body,
        grid=(N // BN,),
        in_specs=[pl.BlockSpec((4, BN, 128), lambda i: (0, i, 0)),
                  full(b1), full(b2), full(w1), full(w2)],
        out_specs=pl.BlockSpec((4, BN, 128), lambda i: (0, i, 0)),
        out_shape=jax.ShapeDtypeStruct((4, N, 128), jnp.float32),
    )(agg, b1, b2, w1, w2)


def _readout_body(agg_ref, b1_ref, b2_ref, r1_ref, rb1_ref, r2_ref, rb2_ref,
                  r3_ref, rb3_ref, out_ref):
    y1 = jnp.maximum(
        jnp.concatenate([agg_ref[0], agg_ref[1]], axis=1) + b1_ref[...], 0.0)
    y2 = jnp.maximum(
        jnp.concatenate([agg_ref[2], agg_ref[3]], axis=1) + b2_ref[...], 0.0)
    y = jnp.concatenate([y1, y2], axis=1).astype(jnp.bfloat16)
    r = _elu(jnp.dot(y, r1_ref[...],
                     preferred_element_type=jnp.float32) + rb1_ref[...])
    r = _elu(jnp.dot(r.astype(jnp.bfloat16), r2_ref[...],
                     preferred_element_type=jnp.float32) + rb2_ref[...])
    out_ref[...] = (jnp.dot(r.astype(jnp.bfloat16), r3_ref[...],
                            preferred_element_type=jnp.float32) + rb3_ref[...])


def _readout(agg, b1, b2, r1, rb1, r2, rb2, r3, rb3):
    full = lambda a: pl.BlockSpec(a.shape, lambda i: (0,) * a.ndim)
    return pl.pallas_call(
        _readout_body,
        grid=(N // BN,),
        in_specs=[pl.BlockSpec((4, BN, 128), lambda i: (0, i, 0)),
                  full(b1), full(b2), full(r1), full(rb1), full(r2), full(rb2),
                  full(r3), full(rb3)],
        out_specs=pl.BlockSpec((BN, 128), lambda i: (i, 0)),
        out_shape=jax.ShapeDtypeStruct((N, 128), jnp.float32),
    )(agg, b1, b2, r1, rb1, r2, rb2, r3, rb3)


def _sc_agg_body(h_hbm, src_hbm, dst_hbm, ew_hbm, out_hbm,
                 src_0, dst_0, ew_0, src_1, dst_1, ew_1,
                 src_2, dst_2, ew_2, src_3, dst_3, ew_3,
                 rows_a, rows_b, acc_sh,
                 isem_0, isem_1, isem_2, isem_3, gsem_a, gsem_b):
    c = lax.axis_index("c")
    s = lax.axis_index("s")
    zero = jnp.zeros((16,), jnp.float32)
    isets = ((src_0, dst_0, ew_0, isem_0), (src_1, dst_1, ew_1, isem_1),
             (src_2, dst_2, ew_2, isem_2), (src_3, dst_3, ew_3, isem_3))
    gslots = ((rows_a, gsem_a), (rows_b, gsem_b))

    n_s = jnp.where(s < NCHUNK % NS, NCHUNK // NS + 1, NCHUNK // NS)
    n_body = (NCHUNK // NS + 1 + 3) // 4

    def zero_row(i, carry):
        for j in range(8):
            rows_a[i, pl.ds(j * 16, 16)] = zero
        return carry

    for k in range(2):
        def idx_copies(q, iset):
            src_v, dst_v, ew_v, isem = iset
            off = (s + q * NS) * EC
            return (
                pltpu.make_async_copy(src_hbm.at[pl.ds(off, EC)], src_v, isem),
                pltpu.make_async_copy(dst_hbm.at[pl.ds(off, EC)], dst_v, isem),
                pltpu.make_async_copy(ew_hbm.at[c].at[pl.ds(off, EC)], ew_v,
                                      isem),
            )

        def load_idx(q, iset):
            for cp in idx_copies(q, iset):
                cp.start()

        def wait_idx_issue_gather(q, iset, gslot):
            for cp in idx_copies(q, iset):
                cp.wait()
            rows_v, gsem = gslot
            pltpu.async_copy(h_hbm.at[2 * c + k].at[iset[0]], rows_v, gsem)

        def wait_scale_scatter(q, iset, gslot):
            src_v, dst_v, ew_v, _ = iset
            rows_v, gsem = gslot
            pltpu.make_async_copy(h_hbm.at[2 * c + k].at[src_v],
                                  rows_v, gsem).wait()

            @plsc.parallel_loop(0, EC // 16, step=1, unroll=2)
            def _(g):
                wg = ew_v[pl.ds(g * 16, 16)]
                for l in range(16):
                    w = jnp.full((16,), wg[l])
                    row = g * 16 + l
                    for j in range(8):
                        rows_v[row, pl.ds(j * 16, 16)] = (
                            rows_v[row, pl.ds(j * 16, 16)] * w)

            pltpu.sync_copy(rows_v, acc_sh.at[dst_v], add=True)

        lax.fori_loop(0, EC, zero_row, 0)
        for r in range(RCHUNK):
            pltpu.sync_copy(rows_a, acc_sh.at[pl.ds(s * RPT + r * EC, EC)])
        plsc.subcore_barrier()

        for q in range(4):
            load_idx(q, isets[q])
        wait_idx_issue_gather(0, isets[0], gslots[0])
        wait_idx_issue_gather(1, isets[1], gslots[1])

        def quad_body(u, carry):
            q0 = 4 * u
            for r in range(4):
                q = q0 + r
                iset, gslot = isets[r], gslots[r % 2]
                iset2 = isets[(r + 2) % 4]

                @pl.when(q < n_s)
                def _():
                    wait_scale_scatter(q, iset, gslot)

                @pl.when(q + 2 < n_s)
                def _():
                    wait_idx_issue_gather(q + 2, iset2, gslot)

                @pl.when(q + 4 < n_s)
                def _():
                    load_idx(q + 4, iset)

            return carry

        lax.fori_loop(0, n_body, quad_body, 0)
        plsc.subcore_barrier()

        pltpu.sync_copy(acc_sh.at[pl.ds(s * RPT, RPT)],
                        out_hbm.at[2 * c + k].at[pl.ds(s * RPT, RPT)])
        plsc.subcore_barrier()


_sc_agg = pl.kernel(
    _sc_agg_body,
    out_type=jax.ShapeDtypeStruct((4, N_PAD, 128), jnp.float32),
    mesh=plsc.VectorSubcoreMesh(core_axis_name="c", subcore_axis_name="s",
                                num_cores=NC, num_subcores=NS),
    scratch_types=(
        [t for _ in range(4)
         for t in (pltpu.VMEM((EC,), jnp.int32),
                   pltpu.VMEM((EC,), jnp.int32),
                   pltpu.VMEM((EC,), jnp.float32))]
        + [
            pltpu.VMEM((EC, 128), jnp.float32),
            pltpu.VMEM((EC, 128), jnp.float32),
            pltpu.VMEM_SHARED((N_PAD, 128), jnp.float32),
            pltpu.SemaphoreType.DMA,
            pltpu.SemaphoreType.DMA,
            pltpu.SemaphoreType.DMA,
            pltpu.SemaphoreType.DMA,
            pltpu.SemaphoreType.DMA,
            pltpu.SemaphoreType.DMA,
        ]),
)


def kernel(x, edge_index, edge_weight, W01, b01, W02, b02, W11, b11, W12, b12,
           W21, b21, W22, b22, R1, rb1, R2, rb2, R3, rb3, Em1, eb1, Em2, eb2,
           Em3, eb3):
    src = edge_index[0]
    dst = edge_index[1]

    bf = lambda a: a.astype(jnp.bfloat16)
    ew = _edge_mlp(edge_weight, Em1, eb1, bf(Em2), eb2, Em3, eb3)

    h = _h0(x, bf(W01), bf(W02))
    agg = _sc_agg(h, src, dst, ew)
    h = _mid(agg, b01, b02, bf(W11), bf(W12))
    agg = _sc_agg(h, src, dst, ew)
    h = _mid(agg, b11, b12, bf(W21), bf(W22))
    agg = _sc_agg(h, src, dst, ew)
    return _readout(agg, b21, b22, bf(R1), rb1, bf(R2), rb2, bf(R3), rb3)
